# Initial kernel scaffold; baseline (speedup 1.0000x reference)
import functools

import jax
import jax.numpy as jnp
from jax import lax
from jax.experimental import pallas as pl
from jax.experimental.pallas import tpu as pltpu
from jax.experimental.pallas import tpu_sc as plsc

NQ, ND, NF = 2048, 16384, 128
M = 32
K = 8
SENT = 1e10
BIG = 3.0e38

BQ, BD = 256, 2048


def _dist_body(x_ref, f_ref, d_ref):
    X = x_ref[...]
    F = f_ref[...]
    mX = jnp.isnan(X)
    mY = jnp.isnan(F)
    mXf = mX.astype(jnp.float32)
    mYf = mY.astype(jnp.float32)
    Xc = jnp.where(mX, 0.0, X)
    Yc = jnp.where(mY, 0.0, F)
    XX = Xc * Xc
    YY = Yc * Yc
    A = jnp.concatenate([-2.0 * Xc, -XX, -mXf], axis=1)
    B = jnp.concatenate([Yc, mYf, YY], axis=1)
    core = lax.dot_general(A, B, (((1,), (1,)), ((), ())),
                           preferred_element_type=jnp.float32)
    xx = XX.sum(axis=1, keepdims=True)
    yy = YY.sum(axis=1).reshape(1, BD)
    nX = mXf.sum(axis=1, keepdims=True)
    nY = mYf.sum(axis=1).reshape(1, BD)
    co_miss = lax.dot_general(mXf, mYf, (((1,), (1,)), ((), ())),
                              preferred_element_type=jnp.float32)
    pres = (float(NF) - nX - nY) + co_miss
    d2 = jnp.maximum(core + xx + yy, 0.0)
    D = jnp.sqrt(d2 / jnp.maximum(pres, 1.0) * float(NF))
    d_ref[...] = jnp.where(pres == 0.0, SENT, D)


def _distances(X, fit_X):
    return pl.pallas_call(
        _dist_body,
        grid=(NQ // BQ, ND // BD),
        in_specs=[
            pl.BlockSpec((BQ, NF), lambda i, j: (i, 0)),
            pl.BlockSpec((BD, NF), lambda i, j: (j, 0)),
        ],
        out_specs=pl.BlockSpec((BQ, BD), lambda i, j: (i, j)),
        out_shape=jax.ShapeDtypeStruct((NQ, ND), jnp.float32),
    )(X, fit_X)


_info = plsc.get_sparse_core_info()
_NC, _NS, _L = _info.num_cores, _info.num_subcores, _info.num_lanes
NW = _NC * _NS
RPW = NQ // NW
NCHUNK = ND // M


def _merge16(k0, i0, k1, i1):
    rk = lax.rev(k1, (0,))
    ri = lax.rev(i1, (0,))
    mlo = k0 <= rk
    lo = jnp.where(mlo, k0, rk)
    loi = jnp.where(mlo, i0, ri)
    hi = jnp.where(mlo, rk, k0)
    hii = jnp.where(mlo, ri, i0)
    c0, ci0 = plsc.sort_key_val(lo, loi)
    c1, ci1 = plsc.sort_key_val(hi, hii)
    return c0, ci0, c1, ci1


def _merge32_low(a0, ai0, a1, ai1, b0, bi0, b1, bi1):
    r0 = lax.rev(b1, (0,))
    r0i = lax.rev(bi1, (0,))
    r1 = lax.rev(b0, (0,))
    r1i = lax.rev(bi0, (0,))
    m0 = a0 <= r0
    lo0 = jnp.where(m0, a0, r0)
    lo0i = jnp.where(m0, ai0, r0i)
    m1 = a1 <= r1
    lo1 = jnp.where(m1, a1, r1)
    lo1i = jnp.where(m1, ai1, r1i)
    me = lo0 <= lo1
    d0 = jnp.where(me, lo0, lo1)
    d0i = jnp.where(me, lo0i, lo1i)
    d1 = jnp.where(me, lo1, lo0)
    d1i = jnp.where(me, lo1i, lo0i)
    k0, i0 = plsc.sort_key_val(d0, d0i)
    k1, i1 = plsc.sort_key_val(d1, d1i)
    return k0, i0, k1, i1


def _topm_body(d_hbm, idx_hbm, dist_hbm, row_v, idx_v, dist_v, sem):
    wid = lax.axis_index("s") * _NC + lax.axis_index("c")
    base = wid * RPW
    lanes = lax.iota(jnp.int32, (_L,))

    def do_row(r, carry_out):
        pltpu.sync_copy(d_hbm.at[base + r], row_v)

        def chunk(j, carry):
            k0, i0, k1, i1, thr = carry
            off = j * M
            a0 = row_v[pl.ds(off, _L)]
            a1 = row_v[pl.ds(off + _L, _L)]
            hit = jnp.any((a0 < thr) | (a1 < thr))

            def insert(c):
                k0, i0, k1, i1, _ = c
                ii0 = off + lanes
                ii1 = off + _L + lanes
                s0, si0 = plsc.sort_key_val(a0, ii0)
                s1, si1 = plsc.sort_key_val(a1, ii1)
                b0, bi0, b1, bi1 = _merge16(s0, si0, s1, si1)
                n0, ni0, n1, ni1 = _merge32_low(k0, i0, k1, i1,
                                                b0, bi0, b1, bi1)
                nthr = lax.reduce_max(n1, axes=(0,))
                return n0, ni0, n1, ni1, nthr

            return lax.cond(hit, insert, lambda c: c,
                            (k0, i0, k1, i1, thr))

        init = (jnp.full((_L,), BIG, jnp.float32),
                jnp.zeros((_L,), jnp.int32),
                jnp.full((_L,), BIG, jnp.float32),
                jnp.zeros((_L,), jnp.int32),
                jnp.float32(BIG))
        k0, i0, k1, i1, _ = lax.fori_loop(0, NCHUNK, chunk, init)
        dist_v[r, pl.ds(0, _L)] = k0
        dist_v[r, pl.ds(_L, _L)] = k1
        idx_v[r, pl.ds(0, _L)] = i0
        idx_v[r, pl.ds(_L, _L)] = i1
        return carry_out

    lax.fori_loop(0, RPW, do_row, 0)
    pltpu.sync_copy(idx_v, idx_hbm.at[pl.ds(base, RPW)])
    pltpu.sync_copy(dist_v, dist_hbm.at[pl.ds(base, RPW)])


def _topm(D):
    mesh = plsc.VectorSubcoreMesh(core_axis_name="c", subcore_axis_name="s")
    return pl.kernel(
        _topm_body,
        out_type=[
            jax.ShapeDtypeStruct((NQ, M), jnp.int32),
            jax.ShapeDtypeStruct((NQ, M), jnp.float32),
        ],
        mesh=mesh,
        scratch_types=[
            pltpu.VMEM((ND,), jnp.float32),
            pltpu.VMEM((RPW, M), jnp.int32),
            pltpu.VMEM((RPW, M), jnp.float32),
            pltpu.SemaphoreType.DMA,
        ],
    )(D)


QB = 16


def _gather_body(fit_hbm, idx_hbm, g_hbm, idx_v, rows_v, sem):
    wid = lax.axis_index("s") * _NC + lax.axis_index("c")
    base = wid * RPW
    pltpu.sync_copy(idx_hbm.at[pl.ds(base, RPW)], idx_v)

    def batch(b, carry):
        pltpu.async_copy(fit_hbm.at[idx_v.at[b]], rows_v, sem).wait()
        pltpu.sync_copy(
            rows_v,
            g_hbm.at[pl.ds((NW * b + wid) * QB, QB)],
        )
        return carry

    lax.fori_loop(0, RPW // QB, batch, 0)


def _gather(fit_X, idx):
    mesh = plsc.VectorSubcoreMesh(core_axis_name="c", subcore_axis_name="s")
    g = pl.kernel(
        _gather_body,
        out_type=jax.ShapeDtypeStruct((NQ // QB, QB, M, NF), jnp.float32),
        mesh=mesh,
        scratch_types=[
            pltpu.VMEM((RPW // QB, QB * M), jnp.int32),
            pltpu.VMEM((QB * M, NF), jnp.float32),
            pltpu.SemaphoreType.DMA,
        ],
    )(fit_X, idx)
    return g


BS = 2048


def _stats_body(f_ref, sum_ref, cnt_ref):
    i = pl.program_id(0)
    F = f_ref[...]
    mY = jnp.isnan(F)
    s = jnp.where(mY, 0.0, F).sum(axis=0, keepdims=True)
    c = (~mY).astype(jnp.float32).sum(axis=0, keepdims=True)

    @pl.when(i == 0)
    def _():
        sum_ref[...] = jnp.zeros_like(sum_ref)
        cnt_ref[...] = jnp.zeros_like(cnt_ref)

    sum_ref[...] += s
    cnt_ref[...] += c


def _stats(fit_X):
    return pl.pallas_call(
        _stats_body,
        grid=(ND // BS,),
        in_specs=[pl.BlockSpec((BS, NF), lambda i: (i, 0))],
        out_specs=[
            pl.BlockSpec((1, NF), lambda i: (0, 0)),
            pl.BlockSpec((1, NF), lambda i: (0, 0)),
        ],
        out_shape=[
            jax.ShapeDtypeStruct((1, NF), jnp.float32),
            jax.ShapeDtypeStruct((1, NF), jnp.float32),
        ],
    )(fit_X)


BQ2 = 64


def _finish_body(x_ref, g_ref, dist_ref, sum_ref, cnt_ref, o_ref):
    X = x_ref[...]
    dist = dist_ref[...]
    cnt = cnt_ref[...]
    colmean = sum_ref[...] / jnp.maximum(cnt, 1.0)

    rank = jnp.zeros((BQ2, NF), jnp.float32)
    num = jnp.zeros((BQ2, NF), jnp.float32)
    den = jnp.zeros((BQ2, NF), jnp.float32)
    for m in range(M):
        g = g_ref[:, m, :]
        p = jnp.logical_not(jnp.isnan(g))
        pf = p.astype(jnp.float32)
        v = jnp.where(p, g, 0.0)
        dfin = (dist[:, m] < 1e9).reshape(BQ2, 1)
        w = jnp.where(p & (rank < float(K)) & dfin, 1.0, 0.0)
        rank = rank + pf
        num = num + w * v
        den = den + w

    val = jnp.where(den > 0.0, num / jnp.maximum(den, 1.0),
                    jnp.broadcast_to(colmean, (BQ2, NF)))
    upd = jnp.isnan(X) & jnp.broadcast_to(cnt > 0.0, (BQ2, NF))
    o_ref[...] = jnp.where(upd, val, X)


def _finish(X, G, dist, sums, cnts):
    return pl.pallas_call(
        _finish_body,
        grid=(NQ // BQ2,),
        in_specs=[
            pl.BlockSpec((BQ2, NF), lambda i: (i, 0)),
            pl.BlockSpec((BQ2, M, NF), lambda i: (i, 0, 0)),
            pl.BlockSpec((BQ2, M), lambda i: (i, 0)),
            pl.BlockSpec((1, NF), lambda i: (0, 0)),
            pl.BlockSpec((1, NF), lambda i: (0, 0)),
        ],
        out_specs=pl.BlockSpec((BQ2, NF), lambda i: (i, 0)),
        out_shape=jax.ShapeDtypeStruct((NQ, NF), jnp.float32),
    )(X, G, dist, sums, cnts)


@jax.jit
def kernel(X, fit_X):
    D = _distances(X, fit_X)
    idx, dist = _topm(D)
    G = _gather(fit_X, idx)
    sums, cnts = _stats(fit_X)
    return _finish(X, G.reshape(NQ, M, NF), dist, sums, cnts)


# speedup vs baseline: 232.9717x; 232.9717x over previous
import jax
import jax.numpy as jnp
from jax import lax
from jax.experimental import pallas as pl
from jax.experimental.pallas import tpu as pltpu
from jax.experimental.pallas import tpu_sc as plsc

NQ, ND, NF = 2048, 16384, 128
M = 32
K = 8
SENT = 1e10
BIG = 3.0e38

BQ, BD = 256, 2048


def _dist_body(x_ref, f_ref, d_ref):
    X = x_ref[...]
    F = f_ref[...]
    mX = jnp.isnan(X)
    mY = jnp.isnan(F)
    mXf = mX.astype(jnp.float32)
    mYf = mY.astype(jnp.float32)
    Xc = jnp.where(mX, 0.0, X)
    Yc = jnp.where(mY, 0.0, F)
    XX = Xc * Xc
    YY = Yc * Yc
    A = jnp.concatenate([-2.0 * Xc, -XX, -mXf], axis=1)
    B = jnp.concatenate([Yc, mYf, YY], axis=1)
    core = lax.dot_general(A, B, (((1,), (1,)), ((), ())),
                           preferred_element_type=jnp.float32)
    xx = XX.sum(axis=1, keepdims=True)
    yy = YY.sum(axis=1).reshape(1, BD)
    nX = mXf.sum(axis=1, keepdims=True)
    nY = mYf.sum(axis=1).reshape(1, BD)
    co_miss = lax.dot_general(mXf, mYf, (((1,), (1,)), ((), ())),
                              preferred_element_type=jnp.float32)
    pres = (float(NF) - nX - nY) + co_miss
    d2 = jnp.maximum(core + xx + yy, 0.0)
    D = jnp.sqrt(d2 / jnp.maximum(pres, 1.0) * float(NF))
    d_ref[...] = jnp.where(pres == 0.0, SENT, D)


def _distances(X, fit_X):
    return pl.pallas_call(
        _dist_body,
        grid=(NQ // BQ, ND // BD),
        in_specs=[
            pl.BlockSpec((BQ, NF), lambda i, j: (i, 0)),
            pl.BlockSpec((BD, NF), lambda i, j: (j, 0)),
        ],
        out_specs=pl.BlockSpec((BQ, BD), lambda i, j: (i, j)),
        out_shape=jax.ShapeDtypeStruct((NQ, ND), jnp.float32),
    )(X, fit_X)


_NC, _NS, _L = 2, 16, 16
NW = _NC * _NS
RPW = NQ // NW
NCHUNK = ND // M


def _merge16(k0, i0, k1, i1):
    rk = lax.rev(k1, (0,))
    ri = lax.rev(i1, (0,))
    mlo = k0 <= rk
    lo = jnp.where(mlo, k0, rk)
    loi = jnp.where(mlo, i0, ri)
    hi = jnp.where(mlo, rk, k0)
    hii = jnp.where(mlo, ri, i0)
    c0, ci0 = plsc.sort_key_val(lo, loi)
    c1, ci1 = plsc.sort_key_val(hi, hii)
    return c0, ci0, c1, ci1


def _merge32_low(a0, ai0, a1, ai1, b0, bi0, b1, bi1):
    r0 = lax.rev(b1, (0,))
    r0i = lax.rev(bi1, (0,))
    r1 = lax.rev(b0, (0,))
    r1i = lax.rev(bi0, (0,))
    m0 = a0 <= r0
    lo0 = jnp.where(m0, a0, r0)
    lo0i = jnp.where(m0, ai0, r0i)
    m1 = a1 <= r1
    lo1 = jnp.where(m1, a1, r1)
    lo1i = jnp.where(m1, ai1, r1i)
    me = lo0 <= lo1
    d0 = jnp.where(me, lo0, lo1)
    d0i = jnp.where(me, lo0i, lo1i)
    d1 = jnp.where(me, lo1, lo0)
    d1i = jnp.where(me, lo1i, lo0i)
    k0, i0 = plsc.sort_key_val(d0, d0i)
    k1, i1 = plsc.sort_key_val(d1, d1i)
    return k0, i0, k1, i1


def _topm_body(d_hbm, idx_hbm, dist_hbm, row_v, idx_v, dist_v):
    wid = lax.axis_index("s") * _NC + lax.axis_index("c")
    base = wid * RPW
    lanes = lax.iota(jnp.int32, _L)

    def do_row(r, carry_out):
        pltpu.sync_copy(d_hbm.at[base + r], row_v)

        def chunk(j, carry):
            k0, i0, k1, i1 = carry
            off = j * M
            a0 = row_v[pl.ds(off, _L)]
            a1 = row_v[pl.ds(off + _L, _L)]
            ii0 = off + lanes
            ii1 = off + _L + lanes
            s0, si0 = plsc.sort_key_val(a0, ii0)
            s1, si1 = plsc.sort_key_val(a1, ii1)
            b0, bi0, b1, bi1 = _merge16(s0, si0, s1, si1)
            return _merge32_low(k0, i0, k1, i1, b0, bi0, b1, bi1)

        init = (jnp.full((_L,), BIG, jnp.float32),
                jnp.zeros((_L,), jnp.int32),
                jnp.full((_L,), BIG, jnp.float32),
                jnp.zeros((_L,), jnp.int32))
        k0, i0, k1, i1 = lax.fori_loop(0, NCHUNK, chunk, init)
        dist_v[r, pl.ds(0, _L)] = k0
        dist_v[r, pl.ds(_L, _L)] = k1
        idx_v[r, pl.ds(0, _L)] = i0
        idx_v[r, pl.ds(_L, _L)] = i1
        return carry_out

    lax.fori_loop(0, RPW, do_row, 0)
    pltpu.sync_copy(idx_v, idx_hbm.at[pl.ds(base, RPW)])
    pltpu.sync_copy(dist_v, dist_hbm.at[pl.ds(base, RPW)])


def _topm(D):
    mesh = plsc.VectorSubcoreMesh(core_axis_name="c", subcore_axis_name="s")
    return pl.kernel(
        _topm_body,
        out_type=[
            jax.ShapeDtypeStruct((NQ, M), jnp.int32),
            jax.ShapeDtypeStruct((NQ, M), jnp.float32),
        ],
        mesh=mesh,
        compiler_params=pltpu.CompilerParams(needs_layout_passes=False),
        scratch_types=[
            pltpu.VMEM((ND,), jnp.float32),
            pltpu.VMEM((RPW, M), jnp.int32),
            pltpu.VMEM((RPW, M), jnp.float32),
        ],
    )(D)


QB = 4
NB = RPW // QB


def _gather_body(fit_hbm, idx_hbm, g_hbm, idx_v, rows_v, sem):
    wid = lax.axis_index("s") * _NC + lax.axis_index("c")
    base = wid * RPW
    pltpu.sync_copy(idx_hbm.at[pl.ds(wid * NB, NB)], idx_v)

    def batch(b, carry):
        pltpu.async_copy(fit_hbm.at[idx_v.at[b]], rows_v, sem).wait()
        pltpu.sync_copy(
            rows_v,
            g_hbm.at[pl.ds((base + b * QB) * M, QB * M)],
        )
        return carry

    lax.fori_loop(0, NB, batch, 0)


def _gather(fit_X, idx):
    mesh = plsc.VectorSubcoreMesh(core_axis_name="c", subcore_axis_name="s")
    g = pl.kernel(
        _gather_body,
        out_type=jax.ShapeDtypeStruct((NQ * M, NF), jnp.float32),
        mesh=mesh,
        compiler_params=pltpu.CompilerParams(needs_layout_passes=False),
        scratch_types=[
            pltpu.VMEM((NB, QB * M), jnp.int32),
            pltpu.VMEM((QB * M, NF), jnp.float32),
            pltpu.SemaphoreType.DMA,
        ],
    )(fit_X, idx.reshape(NQ // QB, QB * M))
    return g


BS = 2048


def _stats_body(f_ref, sum_ref, cnt_ref):
    i = pl.program_id(0)
    F = f_ref[...]
    mY = jnp.isnan(F)
    s = jnp.where(mY, 0.0, F).sum(axis=0, keepdims=True)
    c = (~mY).astype(jnp.float32).sum(axis=0, keepdims=True)

    @pl.when(i == 0)
    def _():
        sum_ref[...] = jnp.zeros_like(sum_ref)
        cnt_ref[...] = jnp.zeros_like(cnt_ref)

    sum_ref[...] += s
    cnt_ref[...] += c


def _stats(fit_X):
    return pl.pallas_call(
        _stats_body,
        grid=(ND // BS,),
        in_specs=[pl.BlockSpec((BS, NF), lambda i: (i, 0))],
        out_specs=[
            pl.BlockSpec((1, NF), lambda i: (0, 0)),
            pl.BlockSpec((1, NF), lambda i: (0, 0)),
        ],
        out_shape=[
            jax.ShapeDtypeStruct((1, NF), jnp.float32),
            jax.ShapeDtypeStruct((1, NF), jnp.float32),
        ],
    )(fit_X)


BQ2 = 64


def _finish_body(x_ref, g_ref, dist_ref, sum_ref, cnt_ref, o_ref):
    X = x_ref[...]
    dist = dist_ref[...]
    cnt = cnt_ref[...]
    colmean = sum_ref[...] / jnp.maximum(cnt, 1.0)

    rank = jnp.zeros((BQ2, NF), jnp.float32)
    num = jnp.zeros((BQ2, NF), jnp.float32)
    den = jnp.zeros((BQ2, NF), jnp.float32)
    for m in range(M):
        g = g_ref[:, m, :]
        p = jnp.logical_not(jnp.isnan(g))
        pf = p.astype(jnp.float32)
        v = jnp.where(p, g, 0.0)
        dfin = (dist[:, m] < 1e9).reshape(BQ2, 1)
        w = jnp.where(p & (rank < float(K)) & dfin, 1.0, 0.0)
        rank = rank + pf
        num = num + w * v
        den = den + w

    val = jnp.where(den > 0.0, num / jnp.maximum(den, 1.0),
                    jnp.broadcast_to(colmean, (BQ2, NF)))
    upd = jnp.isnan(X) & jnp.broadcast_to(cnt > 0.0, (BQ2, NF))
    o_ref[...] = jnp.where(upd, val, X)


def _finish(X, G, dist, sums, cnts):
    return pl.pallas_call(
        _finish_body,
        grid=(NQ // BQ2,),
        in_specs=[
            pl.BlockSpec((BQ2, NF), lambda i: (i, 0)),
            pl.BlockSpec((BQ2, M, NF), lambda i: (i, 0, 0)),
            pl.BlockSpec((BQ2, M), lambda i: (i, 0)),
            pl.BlockSpec((1, NF), lambda i: (0, 0)),
            pl.BlockSpec((1, NF), lambda i: (0, 0)),
        ],
        out_specs=pl.BlockSpec((BQ2, NF), lambda i: (i, 0)),
        out_shape=jax.ShapeDtypeStruct((NQ, NF), jnp.float32),
    )(X, G, dist, sums, cnts)


@jax.jit
def kernel(X, fit_X):
    D = _distances(X, fit_X)
    idx, dist = _topm(D)
    G = _gather(fit_X, idx)
    sums, cnts = _stats(fit_X)
    return _finish(X, G.reshape(NQ, M, NF), dist, sums, cnts)
